# baseline (speedup 1.0000x reference)
import jax
import jax.numpy as jnp
from jax.experimental import pallas as pl
from jax.experimental.pallas import tpu as pltpu

IN_CHANNELS = 384
NUM_EXPERTS = 16
TEMPERATURE = 0.5
HW = 28 * 28
BATCH = 64
HW_BLK = 196


def _router_kernel(x_ref, w_ref, b_ref, o_ref, acc_ref):
    i = pl.program_id(0)
    part = jnp.sum(x_ref[...], axis=0)

    @pl.when(i == 0)
    def _init():
        acc_ref[...] = part

    @pl.when(i > 0)
    def _accum():
        acc_ref[...] += part

    @pl.when(i == pl.num_programs(0) - 1)
    def _finish():
        logits = jax.lax.dot_general(
            acc_ref[...], w_ref[...], (((1,), (1,)), ((), ())),
            preferred_element_type=jnp.float32,
        ) * (1.0 / (HW * TEMPERATURE)) + b_ref[...] * (1.0 / TEMPERATURE)
        m = jnp.max(logits, axis=-1, keepdims=True)
        e = jnp.exp(logits - m)
        o_ref[...] = e / jnp.sum(e, axis=-1, keepdims=True)


def kernel(x, W, b):
    xt = x.transpose(2, 3, 0, 1).reshape(HW, BATCH, IN_CHANNELS)
    out = pl.pallas_call(
        _router_kernel,
        grid=(HW // HW_BLK,),
        in_specs=[
            pl.BlockSpec((HW_BLK, BATCH, IN_CHANNELS), lambda i: (i, 0, 0)),
            pl.BlockSpec((NUM_EXPERTS, IN_CHANNELS), lambda i: (0, 0)),
            pl.BlockSpec((NUM_EXPERTS,), lambda i: (0,)),
        ],
        out_specs=pl.BlockSpec((BATCH, NUM_EXPERTS), lambda i: (0, 0)),
        out_shape=jax.ShapeDtypeStruct((BATCH, NUM_EXPERTS), jnp.float32),
        scratch_shapes=[pltpu.VMEM((BATCH, IN_CHANNELS), jnp.float32)],
    )(xt, W, b)
    return out


# speedup vs baseline: 1.0286x; 1.0286x over previous
import jax
import jax.numpy as jnp
from jax.experimental import pallas as pl
from jax.experimental.pallas import tpu as pltpu

IN_CHANNELS = 384
NUM_EXPERTS = 16
TEMPERATURE = 0.5
HW = 28 * 28
BATCH = 64
HW_BLK = 98


def _router_kernel(x_ref, w_ref, b_ref, o_ref, acc_ref):
    i = pl.program_id(0)
    part = jnp.sum(x_ref[...], axis=0)

    @pl.when(i == 0)
    def _init():
        acc_ref[...] = part

    @pl.when(i > 0)
    def _accum():
        acc_ref[...] += part

    @pl.when(i == pl.num_programs(0) - 1)
    def _finish():
        logits = jax.lax.dot_general(
            acc_ref[...], w_ref[...], (((1,), (1,)), ((), ())),
            preferred_element_type=jnp.float32,
        ) * (1.0 / (HW * TEMPERATURE)) + b_ref[...] * (1.0 / TEMPERATURE)
        m = jnp.max(logits, axis=-1, keepdims=True)
        e = jnp.exp(logits - m)
        o_ref[...] = e / jnp.sum(e, axis=-1, keepdims=True)


def kernel(x, W, b):
    xt = x.transpose(2, 3, 0, 1).reshape(HW, BATCH, IN_CHANNELS)
    out = pl.pallas_call(
        _router_kernel,
        grid=(HW // HW_BLK,),
        in_specs=[
            pl.BlockSpec((HW_BLK, BATCH, IN_CHANNELS), lambda i: (i, 0, 0)),
            pl.BlockSpec((NUM_EXPERTS, IN_CHANNELS), lambda i: (0, 0)),
            pl.BlockSpec((NUM_EXPERTS,), lambda i: (0,)),
        ],
        out_specs=pl.BlockSpec((BATCH, NUM_EXPERTS), lambda i: (0, 0)),
        out_shape=jax.ShapeDtypeStruct((BATCH, NUM_EXPERTS), jnp.float32),
        scratch_shapes=[pltpu.VMEM((BATCH, IN_CHANNELS), jnp.float32)],
    )(xt, W, b)
    return out
